# baseline (speedup 1.0000x reference)
import functools

import jax
import jax.numpy as jnp
from jax.experimental import pallas as pl
from jax.experimental.pallas import tpu as pltpu

_EPS = 1e-5
_TJ = 128


def _body(src_ref, g1_ref, b1_ref, w_in_ref, b_in_ref, cw_ref, cb_ref,
          w_x_ref, w_dt_ref, dtb_ref, atl_ref, dsk_ref, g2_ref, b2_ref,
          w_out_ref, bo_ref, out_ref,
          x3_ref, dt3_ref, dtx3_ref, bc3_ref, cc3_ref, ys3_ref, a_ref,
          *, L, C, Dn, R, N, Wd, TJ):
    Lc = L // TJ

    s = src_ref[0]
    mu = jnp.mean(s, axis=-1, keepdims=True)
    xc = s - mu
    var = jnp.mean(xc * xc, axis=-1, keepdims=True)
    x1 = xc * jax.lax.rsqrt(var + _EPS) * g1_ref[...] + b1_ref[...]
    x2 = jnp.dot(x1, w_in_ref[...],
                 preferred_element_type=jnp.float32) + b_in_ref[...]

    def shift_rows(v, k):
        if k == 0:
            return v
        z = jnp.zeros((abs(k), Dn), jnp.float32)
        if k > 0:
            return jnp.concatenate([v[k:], z], axis=0)
        return jnp.concatenate([z, v[:k]], axis=0)

    w_iota = jax.lax.broadcasted_iota(jnp.int32, (L, 1), 0) % Wd
    m_left = jnp.where(w_iota >= 1, 1.0, 0.0)
    m_right = jnp.where(w_iota <= Wd - 2, 1.0, 0.0)
    acc = None
    for dw in (-1, 0, 1):
        a = None
        for dh in (-1, 0, 1):
            row = (dh + 1) * 3 + (dw + 1)
            term = shift_rows(x2, Wd * dh + dw) * cw_ref[row:row + 1, :]
            a = term if a is None else a + term
        if dw == -1:
            a = a * m_left
        elif dw == 1:
            a = a * m_right
        acc = a if acc is None else acc + a
    xg = jax.nn.gelu(acc + cb_ref[...], approximate=True)

    x3_ref[...] = xg.reshape(Lc, TJ, Dn)
    proj = jnp.dot(xg, w_x_ref[...], preferred_element_type=jnp.float32)
    bc3_ref[...] = proj[:, R:R + N].reshape(Lc, TJ, N)
    cc3_ref[...] = proj[:, R + N:R + 2 * N].reshape(Lc, TJ, N)
    dt = jax.nn.softplus(
        jnp.dot(proj[:, :R], w_dt_ref[...],
                preferred_element_type=jnp.float32) + dtb_ref[...])
    dt3_ref[...] = dt.reshape(Lc, TJ, Dn)
    dtx3_ref[...] = (dt * xg).reshape(Lc, TJ, Dn)
    a_ref[...] = -jnp.exp(atl_ref[...]) * 1.4426950408889634

    Nh = N // 2

    def chunk(c, hpair):
        hA, hB = hpair
        dtc = dt3_ref[c]
        dtxc = dtx3_ref[c]
        bT = bc3_ref[c].T
        cT = cc3_ref[c].T
        for j in range(TJ):
            dt_row = dtc[j:j + 1, :]
            dtx_row = dtxc[j:j + 1, :]
            dAA = jnp.exp2(dt_row * a_ref[:Nh, :])
            dAB = jnp.exp2(dt_row * a_ref[Nh:, :])
            hA = dAA * hA + dtx_row * bT[:Nh, j:j + 1]
            hB = dAB * hB + dtx_row * bT[Nh:, j:j + 1]
            y = (jnp.sum(hA * cT[:Nh, j:j + 1], axis=0, keepdims=True)
                 + jnp.sum(hB * cT[Nh:, j:j + 1], axis=0, keepdims=True))
            ys3_ref[c, j:j + 1, :] = y
        return (hA, hB)

    h0 = jnp.zeros((Nh, Dn), jnp.float32)
    jax.lax.fori_loop(0, Lc, chunk, (h0, h0))

    y = ys3_ref[...].reshape(L, Dn) + x3_ref[...].reshape(L, Dn) * dsk_ref[...]
    mu2 = jnp.mean(y, axis=-1, keepdims=True)
    yc = y - mu2
    var2 = jnp.mean(yc * yc, axis=-1, keepdims=True)
    y2 = yc * jax.lax.rsqrt(var2 + _EPS) * g2_ref[...] + b2_ref[...]
    out_ref[0] = src_ref[0] + jnp.dot(
        y2, w_out_ref[...], preferred_element_type=jnp.float32) + bo_ref[...]


def kernel(src, ln1_g, ln1_b, w_in, b_in, conv_w, conv_b, w_x, w_dt,
           dt_bias, A_log, D_skip, ln2_g, ln2_b, w_out, b_out):
    B, H, W, C = src.shape
    Dn = w_in.shape[1]
    R = w_dt.shape[0]
    N = A_log.shape[1]
    L = H * W
    Lc = L // _TJ

    src2 = src.reshape(B, L, C)
    cw = conv_w.reshape(9, Dn)
    row = lambda v: v.reshape(1, -1)

    body = functools.partial(_body, L=L, C=C, Dn=Dn, R=R, N=N, Wd=W, TJ=_TJ)
    full = lambda shp: pl.BlockSpec(shp, lambda b: (0,) * len(shp))
    out = pl.pallas_call(
        body,
        out_shape=jax.ShapeDtypeStruct((B, L, C), jnp.float32),
        grid=(B,),
        in_specs=[
            pl.BlockSpec((1, L, C), lambda b: (b, 0, 0)),
            full((1, C)), full((1, C)),
            full((C, Dn)), full((1, Dn)),
            full((9, Dn)), full((1, Dn)),
            full((Dn, R + 2 * N)),
            full((R, Dn)), full((1, Dn)),
            full((N, Dn)),
            full((1, Dn)), full((1, Dn)), full((1, Dn)),
            full((Dn, C)), full((1, C)),
        ],
        out_specs=pl.BlockSpec((1, L, C), lambda b: (b, 0, 0)),
        scratch_shapes=[
            pltpu.VMEM((Lc, _TJ, Dn), jnp.float32),
            pltpu.VMEM((Lc, _TJ, Dn), jnp.float32),
            pltpu.VMEM((Lc, _TJ, Dn), jnp.float32),
            pltpu.VMEM((Lc, _TJ, N), jnp.float32),
            pltpu.VMEM((Lc, _TJ, N), jnp.float32),
            pltpu.VMEM((Lc, _TJ, Dn), jnp.float32),
            pltpu.VMEM((N, Dn), jnp.float32),
        ],
        compiler_params=pltpu.CompilerParams(
            dimension_semantics=("arbitrary",),
            vmem_limit_bytes=48 * 1024 * 1024,
        ),
        name="vss_encoder_layer",
    )(src2, row(ln1_g), row(ln1_b), w_in, row(b_in), cw, row(conv_b),
      w_x, w_dt, row(dt_bias), A_log.T, row(D_skip), row(ln2_g),
      row(ln2_b), w_out, row(b_out))
    return out.reshape(B, H, W, C)


# speedup vs baseline: 1.0625x; 1.0625x over previous
import functools

import jax
import jax.numpy as jnp
from jax.experimental import pallas as pl
from jax.experimental.pallas import tpu as pltpu

_EPS = 1e-5
_TJ = 128


def _body(src_ref, g1_ref, b1_ref, w_in_ref, b_in_ref, cw_ref, cb_ref,
          w_x_ref, w_dt_ref, dtb_ref, atl_ref, dsk_ref, g2_ref, b2_ref,
          w_out_ref, bo_ref, out_ref,
          x3_ref, dt3_ref, dtx3_ref, bc3_ref, cc3_ref, ys3_ref, a_ref,
          *, L, C, Dn, R, N, Wd, TJ):
    Lc = L // TJ

    s = src_ref[0]
    mu = jnp.mean(s, axis=-1, keepdims=True)
    xc = s - mu
    var = jnp.mean(xc * xc, axis=-1, keepdims=True)
    x1 = xc * jax.lax.rsqrt(var + _EPS) * g1_ref[...] + b1_ref[...]
    x2 = jnp.dot(x1, w_in_ref[...],
                 preferred_element_type=jnp.float32) + b_in_ref[...]

    def shift_rows(v, k):
        if k == 0:
            return v
        z = jnp.zeros((abs(k), Dn), jnp.float32)
        if k > 0:
            return jnp.concatenate([v[k:], z], axis=0)
        return jnp.concatenate([z, v[:k]], axis=0)

    w_iota = jax.lax.broadcasted_iota(jnp.int32, (L, 1), 0) % Wd
    m_left = jnp.where(w_iota >= 1, 1.0, 0.0)
    m_right = jnp.where(w_iota <= Wd - 2, 1.0, 0.0)
    acc = None
    for dw in (-1, 0, 1):
        a = None
        for dh in (-1, 0, 1):
            row = (dh + 1) * 3 + (dw + 1)
            term = shift_rows(x2, Wd * dh + dw) * cw_ref[row:row + 1, :]
            a = term if a is None else a + term
        if dw == -1:
            a = a * m_left
        elif dw == 1:
            a = a * m_right
        acc = a if acc is None else acc + a
    xg = jax.nn.gelu(acc + cb_ref[...], approximate=True)

    x3_ref[...] = xg.reshape(Lc, TJ, Dn)
    proj = jnp.dot(xg, w_x_ref[...], preferred_element_type=jnp.float32)
    bc3_ref[...] = proj[:, R:R + N].reshape(Lc, TJ, N)
    cc3_ref[...] = proj[:, R + N:R + 2 * N].reshape(Lc, TJ, N)
    dt = jax.nn.softplus(
        jnp.dot(proj[:, :R], w_dt_ref[...],
                preferred_element_type=jnp.float32) + dtb_ref[...])
    dt3_ref[...] = dt.reshape(Lc, TJ, Dn)
    dtx3_ref[...] = (dt * xg).reshape(Lc, TJ, Dn)
    a_ref[...] = -jnp.exp(atl_ref[...]) * 1.4426950408889634

    def chunk(c, h):
        dtc = dt3_ref[c]
        dtxc = dtx3_ref[c]
        bT = bc3_ref[c].T
        cT = cc3_ref[c].T
        for j in range(TJ):
            dA = jnp.exp2(dtc[j:j + 1, :] * a_ref[...])
            u = dtxc[j:j + 1, :] * bT[:, j:j + 1]
            h = dA * h + u
            y = jnp.sum(h * cT[:, j:j + 1], axis=0, keepdims=True)
            ys3_ref[c, j:j + 1, :] = y
        return h

    jax.lax.fori_loop(0, Lc, chunk, jnp.zeros((N, Dn), jnp.float32))

    y = ys3_ref[...].reshape(L, Dn) + x3_ref[...].reshape(L, Dn) * dsk_ref[...]
    mu2 = jnp.mean(y, axis=-1, keepdims=True)
    yc = y - mu2
    var2 = jnp.mean(yc * yc, axis=-1, keepdims=True)
    y2 = yc * jax.lax.rsqrt(var2 + _EPS) * g2_ref[...] + b2_ref[...]
    out_ref[0] = src_ref[0] + jnp.dot(
        y2, w_out_ref[...], preferred_element_type=jnp.float32) + bo_ref[...]


def kernel(src, ln1_g, ln1_b, w_in, b_in, conv_w, conv_b, w_x, w_dt,
           dt_bias, A_log, D_skip, ln2_g, ln2_b, w_out, b_out):
    B, H, W, C = src.shape
    Dn = w_in.shape[1]
    R = w_dt.shape[0]
    N = A_log.shape[1]
    L = H * W
    Lc = L // _TJ

    src2 = src.reshape(B, L, C)
    cw = conv_w.reshape(9, Dn)
    row = lambda v: v.reshape(1, -1)

    body = functools.partial(_body, L=L, C=C, Dn=Dn, R=R, N=N, Wd=W, TJ=_TJ)
    full = lambda shp: pl.BlockSpec(shp, lambda b: (0,) * len(shp))
    out = pl.pallas_call(
        body,
        out_shape=jax.ShapeDtypeStruct((B, L, C), jnp.float32),
        grid=(B,),
        in_specs=[
            pl.BlockSpec((1, L, C), lambda b: (b, 0, 0)),
            full((1, C)), full((1, C)),
            full((C, Dn)), full((1, Dn)),
            full((9, Dn)), full((1, Dn)),
            full((Dn, R + 2 * N)),
            full((R, Dn)), full((1, Dn)),
            full((N, Dn)),
            full((1, Dn)), full((1, Dn)), full((1, Dn)),
            full((Dn, C)), full((1, C)),
        ],
        out_specs=pl.BlockSpec((1, L, C), lambda b: (b, 0, 0)),
        scratch_shapes=[
            pltpu.VMEM((Lc, _TJ, Dn), jnp.float32),
            pltpu.VMEM((Lc, _TJ, Dn), jnp.float32),
            pltpu.VMEM((Lc, _TJ, Dn), jnp.float32),
            pltpu.VMEM((Lc, _TJ, N), jnp.float32),
            pltpu.VMEM((Lc, _TJ, N), jnp.float32),
            pltpu.VMEM((Lc, _TJ, Dn), jnp.float32),
            pltpu.VMEM((N, Dn), jnp.float32),
        ],
        compiler_params=pltpu.CompilerParams(
            dimension_semantics=("arbitrary",),
            vmem_limit_bytes=48 * 1024 * 1024,
        ),
        name="vss_encoder_layer",
    )(src2, row(ln1_g), row(ln1_b), w_in, row(b_in), cw, row(conv_b),
      w_x, w_dt, row(dt_bias), A_log.T, row(D_skip), row(ln2_g),
      row(ln2_b), w_out, row(b_out))
    return out.reshape(B, H, W, C)
